# Initial kernel scaffold; baseline (speedup 1.0000x reference)
import jax
import jax.numpy as jnp
from jax.experimental import pallas as pl


def kernel(queries, values, queries_mask, values_mask, ids, permutation, absolute_positions, relative_positions, pointer_labels, logits_labels, partial_pos, pointer_probs, log_probs, object_detections, object_features, object_boxes, src_table, tgt_table):
    raise NotImplementedError("write your pallas kernel here")


# speedup vs baseline: 2.0726x; 2.0726x over previous
import functools

import jax
import jax.numpy as jnp
from jax import lax
from jax.experimental import pallas as pl
from jax.experimental.pallas import tpu as pltpu
from jax.experimental.pallas import tpu_sc as plsc


def _pos_encoding(length, hidden_size):
    pos = jnp.arange(length, dtype=jnp.float32)[:, None]
    dims = jnp.arange(hidden_size, dtype=jnp.float32)[None, :]
    angle_rates = jnp.power(10000.0, -2.0 * jnp.floor(dims / 2.0) / float(hidden_size))
    angles = pos * angle_rates
    even = (jnp.arange(hidden_size)[None, :] % 2) == 0
    return jnp.where(even, jnp.sin(angles), jnp.cos(angles))


@functools.lru_cache(maxsize=None)
def _make_sc_gather(n_rows, hidden, chunk):
    info = plsc.get_sparse_core_info()
    nc, ns = info.num_cores, info.num_subcores
    nw = nc * ns
    per_w = n_rows // nw
    assert n_rows % nw == 0 and per_w % chunk == 0
    n_ch = per_w // chunk

    mesh = plsc.VectorSubcoreMesh(core_axis_name="c", subcore_axis_name="s")

    @functools.partial(
        pl.kernel,
        mesh=mesh,
        compiler_params=pltpu.CompilerParams(use_tc_tiling_on_sc=False),
        out_type=(
            jax.ShapeDtypeStruct((n_rows, hidden), jnp.float32),
            jax.ShapeDtypeStruct((n_rows, hidden), jnp.float32),
        ),
        scratch_types=[
            pltpu.VMEM((chunk,), jnp.int32),
            pltpu.VMEM((chunk, hidden), jnp.float32),
            pltpu.SemaphoreType.DMA,
        ],
    )
    def sc_kernel(q_idx, v_idx, tgt, src, eq, ev, idx_v, rows_v, sem):
        wid = lax.axis_index("s") * nc + lax.axis_index("c")
        base = wid * per_w

        def gather_one(idx_hbm, table, out):
            def step(i, carry):
                off = base + i * chunk
                pltpu.sync_copy(idx_hbm.at[pl.ds(off, chunk)], idx_v)
                pltpu.async_copy(table.at[idx_v], rows_v, sem).wait()
                pltpu.sync_copy(rows_v, out.at[pl.ds(off, chunk)])
                return carry

            lax.fori_loop(0, n_ch, step, 0)

        gather_one(q_idx, tgt, eq)
        gather_one(v_idx, src, ev)

    return sc_kernel


def _tc_body(g, a_ref, eq_ref, ev_ref, peq_ref, pev_ref, b_ref, d_ref):
    peq = peq_ref[...]
    for i in range(g):
        b_ref[i, :, :] = peq + jnp.dot(
            a_ref[i, :, :], eq_ref[i, :, :], preferred_element_type=jnp.float32
        )
    d_ref[...] = ev_ref[...] + pev_ref[...][None]


@functools.lru_cache(maxsize=None)
def _make_tc(batch, lq, lv, hidden, g):
    assert batch % g == 0
    return pl.pallas_call(
        functools.partial(_tc_body, g),
        grid=(batch // g,),
        in_specs=[
            pl.BlockSpec((g, lq, lq), lambda i: (i, 0, 0)),
            pl.BlockSpec((g, lq, hidden), lambda i: (i, 0, 0)),
            pl.BlockSpec((g, lv, hidden), lambda i: (i, 0, 0)),
            pl.BlockSpec((lq, hidden), lambda i: (0, 0)),
            pl.BlockSpec((lv, hidden), lambda i: (0, 0)),
        ],
        out_specs=[
            pl.BlockSpec((g, lq, hidden), lambda i: (i, 0, 0)),
            pl.BlockSpec((g, lv, hidden), lambda i: (i, 0, 0)),
        ],
        out_shape=[
            jax.ShapeDtypeStruct((batch, lq, hidden), jnp.float32),
            jax.ShapeDtypeStruct((batch, lv, hidden), jnp.float32),
        ],
    )


def kernel(queries, values, queries_mask, values_mask, ids, permutation,
           absolute_positions, relative_positions, pointer_labels,
           logits_labels, partial_pos, pointer_probs, log_probs,
           object_detections, object_features, object_boxes,
           src_table, tgt_table):
    batch, lq = queries.shape
    lv = values.shape[1]
    hidden = tgt_table.shape[1]

    q_flat = queries.reshape(-1).astype(jnp.int32)
    v_flat = values.reshape(-1).astype(jnp.int32)

    sc = _make_sc_gather(batch * lq, hidden, 1280)
    eq, ev = sc(q_flat, v_flat, tgt_table, src_table)

    peq = _pos_encoding(lq, hidden)
    pev = _pos_encoding(lv, hidden)

    tc = _make_tc(batch, lq, lv, hidden, 8)
    b, d = tc(
        absolute_positions,
        eq.reshape(batch, lq, hidden),
        ev.reshape(batch, lv, hidden),
        peq,
        pev,
    )

    return (b, d, queries_mask, values_mask, ids, permutation,
            absolute_positions, relative_positions, pointer_labels,
            logits_labels, partial_pos, pointer_probs, log_probs,
            object_detections, object_features, object_boxes)


# speedup vs baseline: 2.1016x; 1.0140x over previous
import functools

import jax
import jax.numpy as jnp
from jax import lax
from jax.experimental import pallas as pl
from jax.experimental.pallas import tpu as pltpu
from jax.experimental.pallas import tpu_sc as plsc


def _pos_encoding(length, hidden_size):
    pos = jnp.arange(length, dtype=jnp.float32)[:, None]
    dims = jnp.arange(hidden_size, dtype=jnp.float32)[None, :]
    angle_rates = jnp.power(10000.0, -2.0 * jnp.floor(dims / 2.0) / float(hidden_size))
    angles = pos * angle_rates
    even = (jnp.arange(hidden_size)[None, :] % 2) == 0
    return jnp.where(even, jnp.sin(angles), jnp.cos(angles))


@functools.lru_cache(maxsize=None)
def _make_sc_gather(n_rows, hidden, chunk):
    info = plsc.get_sparse_core_info()
    nc, ns = info.num_cores, info.num_subcores
    nw = nc * ns
    per_w = n_rows // nw
    assert n_rows % nw == 0 and per_w % chunk == 0
    n_ch = per_w // chunk

    mesh = plsc.VectorSubcoreMesh(core_axis_name="c", subcore_axis_name="s")

    @functools.partial(
        pl.kernel,
        mesh=mesh,
        compiler_params=pltpu.CompilerParams(use_tc_tiling_on_sc=False),
        out_type=(
            jax.ShapeDtypeStruct((n_rows, hidden), jnp.float32),
            jax.ShapeDtypeStruct((n_rows, hidden), jnp.float32),
        ),
        scratch_types=[
            pltpu.VMEM((chunk,), jnp.int32),
            pltpu.VMEM((chunk, hidden), jnp.float32),
            pltpu.SemaphoreType.DMA,
        ],
    )
    def sc_kernel(q_idx, v_idx, tgt, src, eq, ev, idx_v, rows_v, sem):
        wid = lax.axis_index("s") * nc + lax.axis_index("c")
        base = wid * per_w

        def gather_one(idx_hbm, table, out):
            def step(i, carry):
                off = base + i * chunk
                pltpu.sync_copy(idx_hbm.at[pl.ds(off, chunk)], idx_v)
                pltpu.async_copy(table.at[idx_v], rows_v, sem).wait()
                pltpu.sync_copy(rows_v, out.at[pl.ds(off, chunk)])
                return carry

            lax.fori_loop(0, n_ch, step, 0)

        gather_one(q_idx, tgt, eq)
        gather_one(v_idx, src, ev)

    return sc_kernel


def _tc_body(g, lq, lv, a_ref, eq_ref, ev_ref, peq_ref, pev_ref, b_ref, d_ref):
    peq = peq_ref[...]
    pev = pev_ref[...]
    for i in range(g):
        b_ref[i, :, :] = peq + jnp.dot(
            a_ref[i, :, :], eq_ref[pl.ds(i * lq, lq), :],
            preferred_element_type=jnp.float32,
        )
        d_ref[i, :, :] = pev + ev_ref[pl.ds(i * lv, lv), :]


@functools.lru_cache(maxsize=None)
def _make_tc(batch, lq, lv, hidden, g):
    assert batch % g == 0
    return pl.pallas_call(
        functools.partial(_tc_body, g, lq, lv),
        grid=(batch // g,),
        in_specs=[
            pl.BlockSpec((g, lq, lq), lambda i: (i, 0, 0)),
            pl.BlockSpec((g * lq, hidden), lambda i: (i, 0)),
            pl.BlockSpec((g * lv, hidden), lambda i: (i, 0)),
            pl.BlockSpec((lq, hidden), lambda i: (0, 0)),
            pl.BlockSpec((lv, hidden), lambda i: (0, 0)),
        ],
        out_specs=[
            pl.BlockSpec((g, lq, hidden), lambda i: (i, 0, 0)),
            pl.BlockSpec((g, lv, hidden), lambda i: (i, 0, 0)),
        ],
        out_shape=[
            jax.ShapeDtypeStruct((batch, lq, hidden), jnp.float32),
            jax.ShapeDtypeStruct((batch, lv, hidden), jnp.float32),
        ],
    )


def kernel(queries, values, queries_mask, values_mask, ids, permutation,
           absolute_positions, relative_positions, pointer_labels,
           logits_labels, partial_pos, pointer_probs, log_probs,
           object_detections, object_features, object_boxes,
           src_table, tgt_table):
    batch, lq = queries.shape
    lv = values.shape[1]
    hidden = tgt_table.shape[1]

    q_flat = queries.reshape(-1).astype(jnp.int32)
    v_flat = values.reshape(-1).astype(jnp.int32)

    sc = _make_sc_gather(batch * lq, hidden, 1280)
    eq, ev = sc(q_flat, v_flat, tgt_table, src_table)

    peq = _pos_encoding(lq, hidden)
    pev = _pos_encoding(lv, hidden)

    tc = _make_tc(batch, lq, lv, hidden, 8)
    b, d = tc(absolute_positions, eq, ev, peq, pev)

    return (b, d, queries_mask, values_mask, ids, permutation,
            absolute_positions, relative_positions, pointer_labels,
            logits_labels, partial_pos, pointer_probs, log_probs,
            object_detections, object_features, object_boxes)


# speedup vs baseline: 2.6079x; 1.2409x over previous
import functools

import jax
import jax.numpy as jnp
from jax import lax
from jax.experimental import pallas as pl
from jax.experimental.pallas import tpu as pltpu
from jax.experimental.pallas import tpu_sc as plsc


def _pos_encoding(length, hidden_size):
    pos = jnp.arange(length, dtype=jnp.float32)[:, None]
    dims = jnp.arange(hidden_size, dtype=jnp.float32)[None, :]
    angle_rates = jnp.power(10000.0, -2.0 * jnp.floor(dims / 2.0) / float(hidden_size))
    angles = pos * angle_rates
    even = (jnp.arange(hidden_size)[None, :] % 2) == 0
    return jnp.where(even, jnp.sin(angles), jnp.cos(angles))


@functools.lru_cache(maxsize=None)
def _make_sc_gather(n_rows, hidden, chunk):
    info = plsc.get_sparse_core_info()
    nc, ns = info.num_cores, info.num_subcores
    nw = nc * ns
    per_w = n_rows // nw
    assert n_rows % nw == 0 and per_w % chunk == 0
    n_ch = per_w // chunk

    mesh = plsc.VectorSubcoreMesh(core_axis_name="c", subcore_axis_name="s")

    @functools.partial(
        pl.kernel,
        mesh=mesh,
        compiler_params=pltpu.CompilerParams(use_tc_tiling_on_sc=False),
        out_type=(
            jax.ShapeDtypeStruct((n_rows, hidden), jnp.float32),
            jax.ShapeDtypeStruct((n_rows, hidden), jnp.float32),
        ),
        scratch_types=[
            pltpu.VMEM((chunk,), jnp.int32),
            pltpu.VMEM((chunk, hidden), jnp.float32),
            pltpu.SemaphoreType.DMA,
        ],
    )
    def sc_kernel(q_idx, v_idx, tgt, src, eq, ev, idx_v, rows_v, sem):
        wid = lax.axis_index("s") * nc + lax.axis_index("c")
        base = wid * per_w

        def gather_one(idx_hbm, table, out):
            def step(i, carry):
                off = base + i * chunk
                pltpu.sync_copy(idx_hbm.at[pl.ds(off, chunk)], idx_v)
                pltpu.async_copy(table.at[idx_v], rows_v, sem).wait()
                pltpu.sync_copy(rows_v, out.at[pl.ds(off, chunk)])
                return carry

            lax.fori_loop(0, n_ch, step, 0)

        gather_one(q_idx, tgt, eq)
        gather_one(v_idx, src, ev)

    return sc_kernel


def _tc_body(g, lq, lv, a_ref, eq_ref, ev_ref, peq_ref, pev_ref, b_ref, d_ref):
    peq = peq_ref[...]
    pev = pev_ref[...]
    for i in range(g):
        b_ref[i, :, :] = peq + jnp.dot(
            a_ref[i, :, :], eq_ref[pl.ds(i * lq, lq), :],
            preferred_element_type=jnp.float32,
        )
        d_ref[i, :, :] = pev + ev_ref[pl.ds(i * lv, lv), :]


@functools.lru_cache(maxsize=None)
def _make_tc(batch, lq, lv, hidden, g):
    assert batch % g == 0
    return pl.pallas_call(
        functools.partial(_tc_body, g, lq, lv),
        grid=(batch // g,),
        in_specs=[
            pl.BlockSpec((g, lq, lq), lambda i: (i, 0, 0)),
            pl.BlockSpec((g * lq, hidden), lambda i: (i, 0)),
            pl.BlockSpec((g * lv, hidden), lambda i: (i, 0)),
            pl.BlockSpec((lq, hidden), lambda i: (0, 0)),
            pl.BlockSpec((lv, hidden), lambda i: (0, 0)),
        ],
        out_specs=[
            pl.BlockSpec((g, lq, hidden), lambda i: (i, 0, 0)),
            pl.BlockSpec((g, lv, hidden), lambda i: (i, 0, 0)),
        ],
        out_shape=[
            jax.ShapeDtypeStruct((batch, lq, hidden), jnp.float32),
            jax.ShapeDtypeStruct((batch, lv, hidden), jnp.float32),
        ],
    )


def kernel(queries, values, queries_mask, values_mask, ids, permutation,
           absolute_positions, relative_positions, pointer_labels,
           logits_labels, partial_pos, pointer_probs, log_probs,
           object_detections, object_features, object_boxes,
           src_table, tgt_table):
    batch, lq = queries.shape
    lv = values.shape[1]
    hidden = tgt_table.shape[1]

    q_flat = queries.reshape(-1).astype(jnp.int32)
    v_flat = values.reshape(-1).astype(jnp.int32)

    sc = _make_sc_gather(batch * lq, hidden, 1280)
    eq, ev = sc(q_flat, v_flat, tgt_table, src_table)

    peq = _pos_encoding(lq, hidden)
    pev = _pos_encoding(lv, hidden)

    tc = _make_tc(batch, lq, lv, hidden, 32)
    b, d = tc(absolute_positions, eq, ev, peq, pev)

    return (b, d, queries_mask, values_mask, ids, permutation,
            absolute_positions, relative_positions, pointer_labels,
            logits_labels, partial_pos, pointer_probs, log_probs,
            object_detections, object_features, object_boxes)


# speedup vs baseline: 2.6667x; 1.0225x over previous
import functools

import jax
import jax.numpy as jnp
from jax import lax
from jax.experimental import pallas as pl
from jax.experimental.pallas import tpu as pltpu
from jax.experimental.pallas import tpu_sc as plsc


def _pos_encoding(length, hidden_size):
    pos = jnp.arange(length, dtype=jnp.float32)[:, None]
    dims = jnp.arange(hidden_size, dtype=jnp.float32)[None, :]
    angle_rates = jnp.power(10000.0, -2.0 * jnp.floor(dims / 2.0) / float(hidden_size))
    angles = pos * angle_rates
    even = (jnp.arange(hidden_size)[None, :] % 2) == 0
    return jnp.where(even, jnp.sin(angles), jnp.cos(angles))


@functools.lru_cache(maxsize=None)
def _make_sc_gather(n_rows, hidden, chunk):
    info = plsc.get_sparse_core_info()
    nc, ns = info.num_cores, info.num_subcores
    nw = nc * ns
    per_w = n_rows // nw
    assert n_rows % nw == 0 and per_w % chunk == 0
    n_ch = per_w // chunk

    mesh = plsc.VectorSubcoreMesh(core_axis_name="c", subcore_axis_name="s")

    @functools.partial(
        pl.kernel,
        mesh=mesh,
        compiler_params=pltpu.CompilerParams(use_tc_tiling_on_sc=False),
        out_type=(
            jax.ShapeDtypeStruct((n_rows, hidden), jnp.float32),
            jax.ShapeDtypeStruct((n_rows, hidden), jnp.float32),
        ),
        scratch_types=[
            pltpu.VMEM((chunk,), jnp.int32),
            pltpu.VMEM((chunk, hidden), jnp.float32),
            pltpu.SemaphoreType.DMA,
        ],
    )
    def sc_kernel(q_idx, v_idx, tgt, src, eq, ev, idx_v, rows_v, sem):
        wid = lax.axis_index("s") * nc + lax.axis_index("c")
        base = wid * per_w

        def gather_one(idx_hbm, table, out):
            def step(i, carry):
                off = base + i * chunk
                pltpu.sync_copy(idx_hbm.at[pl.ds(off, chunk)], idx_v)
                pltpu.async_copy(table.at[idx_v], rows_v, sem).wait()
                pltpu.sync_copy(rows_v, out.at[pl.ds(off, chunk)])
                return carry

            lax.fori_loop(0, n_ch, step, 0)

        gather_one(q_idx, tgt, eq)
        gather_one(v_idx, src, ev)

    return sc_kernel


def _tc_body(g, lq, lv, a_ref, eq_ref, ev_ref, peq_ref, pev_ref, b_ref, d_ref):
    peq = peq_ref[...]
    pev = pev_ref[...]
    for i in range(g):
        b_ref[i, :, :] = peq + jnp.dot(
            a_ref[i, :, :], eq_ref[pl.ds(i * lq, lq), :],
            preferred_element_type=jnp.float32,
        )
        d_ref[i, :, :] = pev + ev_ref[pl.ds(i * lv, lv), :]


@functools.lru_cache(maxsize=None)
def _make_tc(batch, lq, lv, hidden, g):
    assert batch % g == 0
    return pl.pallas_call(
        functools.partial(_tc_body, g, lq, lv),
        grid=(batch // g,),
        in_specs=[
            pl.BlockSpec((g, lq, lq), lambda i: (i, 0, 0)),
            pl.BlockSpec((g * lq, hidden), lambda i: (i, 0)),
            pl.BlockSpec((g * lv, hidden), lambda i: (i, 0)),
            pl.BlockSpec((lq, hidden), lambda i: (0, 0)),
            pl.BlockSpec((lv, hidden), lambda i: (0, 0)),
        ],
        out_specs=[
            pl.BlockSpec((g, lq, hidden), lambda i: (i, 0, 0)),
            pl.BlockSpec((g, lv, hidden), lambda i: (i, 0, 0)),
        ],
        out_shape=[
            jax.ShapeDtypeStruct((batch, lq, hidden), jnp.float32),
            jax.ShapeDtypeStruct((batch, lv, hidden), jnp.float32),
        ],
    )


def kernel(queries, values, queries_mask, values_mask, ids, permutation,
           absolute_positions, relative_positions, pointer_labels,
           logits_labels, partial_pos, pointer_probs, log_probs,
           object_detections, object_features, object_boxes,
           src_table, tgt_table):
    batch, lq = queries.shape
    lv = values.shape[1]
    hidden = tgt_table.shape[1]

    q_flat = queries.reshape(-1).astype(jnp.int32)
    v_flat = values.reshape(-1).astype(jnp.int32)

    sc = _make_sc_gather(batch * lq, hidden, 1280)
    eq, ev = sc(q_flat, v_flat, tgt_table, src_table)

    peq = _pos_encoding(lq, hidden)
    pev = _pos_encoding(lv, hidden)

    tc = _make_tc(batch, lq, lv, hidden, 64)
    b, d = tc(absolute_positions, eq, ev, peq, pev)

    return (b, d, queries_mask, values_mask, ids, permutation,
            absolute_positions, relative_positions, pointer_labels,
            logits_labels, partial_pos, pointer_probs, log_probs,
            object_detections, object_features, object_boxes)
